# baseline (speedup 1.0000x reference)
import jax
import jax.numpy as jnp
from jax.experimental import pallas as pl
from jax.experimental.pallas import tpu as pltpu

_BM = 512


def _cast(x_ref, dst_ref):
    dst_ref[...] = x_ref[...].astype(jnp.bfloat16)


def _dot(src_ref, wb_ref, o_ref):
    acc = jnp.dot(src_ref[...], wb_ref[...], preferred_element_type=jnp.float32)
    o_ref[...] = jnp.maximum(acc, 0.0)


def _mm_relu(x_ref, w_hbm, o_ref, wf0_ref, wf1_ref, wb_ref, xa_ref, xb_ref,
             sem0, sem1):
    K, N = w_hbm.shape
    nh = N // 2
    i = pl.program_id(0)

    @pl.when(i == 0)
    def _():
        cp0 = pltpu.make_async_copy(w_hbm.at[:, pl.ds(0, nh)], wf0_ref, sem0)
        cp1 = pltpu.make_async_copy(w_hbm.at[:, pl.ds(nh, nh)], wf1_ref, sem1)
        cp0.start()
        cp1.start()
        _cast(x_ref, xa_ref)

    @pl.when(i == 1)
    def _():
        cp0 = pltpu.make_async_copy(w_hbm.at[:, pl.ds(0, nh)], wf0_ref, sem0)
        cp1 = pltpu.make_async_copy(w_hbm.at[:, pl.ds(nh, nh)], wf1_ref, sem1)
        cp0.wait()
        wb_ref[:, :nh] = wf0_ref[...].astype(jnp.bfloat16)
        acc0 = jnp.dot(xa_ref[...], wb_ref[:, :nh],
                       preferred_element_type=jnp.float32)
        o_ref[:, :nh] = jnp.maximum(acc0, 0.0)
        cp1.wait()
        wb_ref[:, nh:] = wf1_ref[...].astype(jnp.bfloat16)
        acc1 = jnp.dot(xa_ref[...], wb_ref[:, nh:],
                       preferred_element_type=jnp.float32)
        o_ref[:, nh:] = jnp.maximum(acc1, 0.0)
        _cast(x_ref, xb_ref)

    @pl.when(jnp.logical_and(i > 1, i % 2 == 0))
    def _():
        _cast(x_ref, xa_ref)
        _dot(xb_ref, wb_ref, o_ref)

    @pl.when(jnp.logical_and(i > 1, i % 2 == 1))
    def _():
        _cast(x_ref, xb_ref)
        _dot(xa_ref, wb_ref, o_ref)


def kernel(x_affine, W):
    M, K = x_affine.shape
    _, N = W.shape
    nblk = M // _BM
    return pl.pallas_call(
        _mm_relu,
        grid=(nblk + 1,),
        in_specs=[
            pl.BlockSpec((_BM, K), lambda i: (jnp.minimum(i, nblk - 1), 0)),
            pl.BlockSpec(memory_space=pl.ANY),
        ],
        out_specs=pl.BlockSpec((_BM, N), lambda i: (jnp.maximum(i - 1, 0), 0)),
        out_shape=jax.ShapeDtypeStruct((M, N), jnp.float32),
        scratch_shapes=[
            pltpu.VMEM((K, N // 2), jnp.float32),
            pltpu.VMEM((K, N // 2), jnp.float32),
            pltpu.VMEM((K, N), jnp.bfloat16),
            pltpu.VMEM((_BM, K), jnp.bfloat16),
            pltpu.VMEM((_BM, K), jnp.bfloat16),
            pltpu.SemaphoreType.DMA,
            pltpu.SemaphoreType.DMA,
        ],
        compiler_params=pltpu.CompilerParams(
            dimension_semantics=("arbitrary",),
        ),
    )(x_affine, W)


# speedup vs baseline: 1.0853x; 1.0853x over previous
import jax
import jax.numpy as jnp
from jax.experimental import pallas as pl
from jax.experimental.pallas import tpu as pltpu

_BM = 512


def _mm_relu(x_ref, w_hbm, o_ref, wf0_ref, wf1_ref, wb_ref, sem0, sem1):
    K, N = w_hbm.shape
    nh = N // 2
    i = pl.program_id(0)

    @pl.when(i == 0)
    def _():
        cp0 = pltpu.make_async_copy(w_hbm.at[:, pl.ds(0, nh)], wf0_ref, sem0)
        cp1 = pltpu.make_async_copy(w_hbm.at[:, pl.ds(nh, nh)], wf1_ref, sem1)
        cp0.start()
        cp1.start()
        xb = x_ref[...].astype(jnp.bfloat16)
        cp0.wait()
        wb_ref[:, :nh] = wf0_ref[...].astype(jnp.bfloat16)
        acc0 = jnp.dot(xb, wb_ref[:, :nh], preferred_element_type=jnp.float32)
        o_ref[:, :nh] = jnp.maximum(acc0, 0.0)
        cp1.wait()
        wb_ref[:, nh:] = wf1_ref[...].astype(jnp.bfloat16)
        acc1 = jnp.dot(xb, wb_ref[:, nh:], preferred_element_type=jnp.float32)
        o_ref[:, nh:] = jnp.maximum(acc1, 0.0)

    @pl.when(i > 0)
    def _():
        xb = x_ref[...].astype(jnp.bfloat16)
        acc = jnp.dot(xb, wb_ref[...], preferred_element_type=jnp.float32)
        o_ref[...] = jnp.maximum(acc, 0.0)


def kernel(x_affine, W):
    M, K = x_affine.shape
    _, N = W.shape
    return pl.pallas_call(
        _mm_relu,
        grid=(M // _BM,),
        in_specs=[
            pl.BlockSpec((_BM, K), lambda i: (i, 0)),
            pl.BlockSpec(memory_space=pl.ANY),
        ],
        out_specs=pl.BlockSpec((_BM, N), lambda i: (i, 0)),
        out_shape=jax.ShapeDtypeStruct((M, N), jnp.float32),
        scratch_shapes=[
            pltpu.VMEM((K, N // 2), jnp.float32),
            pltpu.VMEM((K, N // 2), jnp.float32),
            pltpu.VMEM((K, N), jnp.bfloat16),
            pltpu.SemaphoreType.DMA,
            pltpu.SemaphoreType.DMA,
        ],
        compiler_params=pltpu.CompilerParams(
            dimension_semantics=("arbitrary",),
        ),
    )(x_affine, W)
